# Initial kernel scaffold; baseline (speedup 1.0000x reference)
import jax
import jax.numpy as jnp
from jax.experimental import pallas as pl


def kernel(x, emb_tables, lin_tables, W1, b1, W2, b2, Wf, bf):
    raise NotImplementedError("write your pallas kernel here")


# speedup vs baseline: 1.1720x; 1.1720x over previous
import functools

import jax
import jax.numpy as jnp
from jax import lax
from jax.experimental import pallas as pl
from jax.experimental.pallas import tpu as pltpu
from jax.experimental.pallas import tpu_sc as plsc

F = 26
V = 100000
D = 32
B = 4096
H1 = 256
H2 = 128

NC = 2
NS = 16
NW = NC * NS
BPW = B // NW
LPW = BPW * F
CHUNK = BPW
NBUF = 4

_SC_MESH = plsc.VectorSubcoreMesh(core_axis_name="c", subcore_axis_name="s")


LG = 32


@functools.partial(
    pl.kernel,
    out_type=(
        jax.ShapeDtypeStruct((F, NW, CHUNK, D), jnp.float32),
        jax.ShapeDtypeStruct((NW, LPW), jnp.float32),
    ),
    mesh=_SC_MESH,
    scratch_types=(
        pltpu.VMEM((F, CHUNK), jnp.int32),
        pltpu.VMEM((F, CHUNK), jnp.int32),
        pltpu.VMEM((F, CHUNK), jnp.int32),
        pltpu.VMEM((NBUF, CHUNK, D), jnp.float32),
        pltpu.VMEM((NBUF, CHUNK, LG), jnp.float32),
        pltpu.VMEM((LPW,), jnp.float32),
        pltpu.SemaphoreType.DMA((NBUF,)),
        pltpu.SemaphoreType.DMA((NBUF,)),
        pltpu.SemaphoreType.DMA((NBUF,)),
        pltpu.SemaphoreType.DMA,
    ),
    compiler_params=pltpu.CompilerParams(use_tc_tiling_on_sc=False,
                                         needs_layout_passes=False),
)
def _sc_gather(xw, xq, xr, emb3, lin32, out_emb, out_lin, idx_v, idxq_v,
               rem_v, ring, lring, lin_v, sem_g, sem_o, sem_lg, sem_lo):
    wid = lax.axis_index("s") * NC + lax.axis_index("c")
    pltpu.sync_copy(xw.at[wid], idx_v)
    pltpu.sync_copy(xq.at[wid], idxq_v)
    pltpu.sync_copy(xr.at[wid], rem_v)

    def extract_lin(field, slot):
        for g in range(CHUNK // 16):
            rows = lax.iota(jnp.int32, 16) + (g * 16)
            lanes = rem_v[field, pl.ds(g * 16, 16)]
            vals = plsc.load_gather(lring.at[slot], [rows, lanes])
            lin_v[pl.ds(field * CHUNK + g * 16, 16)] = vals

    g_cps = [None] * NBUF
    o_cps = [None] * NBUF
    l_cps = [None] * NBUF
    for f in range(F):
        s = f % NBUF
        if f >= NBUF:
            o_cps[s].wait()
        g_cps[s] = pltpu.async_copy(emb3.at[f].at[idx_v.at[f]], ring.at[s],
                                    sem_g.at[s])
        l_cps[s] = pltpu.async_copy(lin32.at[f].at[idxq_v.at[f]],
                                    lring.at[s], sem_lg.at[s])
        d = f - (NBUF - 1)
        if d >= 0:
            sd = d % NBUF
            g_cps[sd].wait()
            o_cps[sd] = pltpu.async_copy(ring.at[sd], out_emb.at[d, wid],
                                         sem_o.at[sd])
            l_cps[sd].wait()
            extract_lin(d, sd)
    for d in range(F - NBUF + 1, F):
        sd = d % NBUF
        g_cps[sd].wait()
        o_cps[sd] = pltpu.async_copy(ring.at[sd], out_emb.at[d, wid],
                                     sem_o.at[sd])
        l_cps[sd].wait()
        extract_lin(d, sd)
    for cp in o_cps:
        cp.wait()
    pltpu.sync_copy(lin_v, out_lin.at[wid])


BB = 512


def _tc_body(scal_ref, e_ref, lin_ref, w1_ref, b1_ref, w2_ref, b2_ref,
             wfh_ref, o_ref):
    s = jnp.zeros((BB, D), jnp.float32)
    ssq = jnp.zeros((BB, 1), jnp.float32)
    h = jnp.zeros((BB, H1), jnp.float32)
    for f in range(F):
        ef = e_ref[f]
        s = s + ef
        ssq = ssq + jnp.sum(ef * ef, axis=1, keepdims=True)
        h = h + jnp.dot(ef, w1_ref[f], preferred_element_type=jnp.float32)
    second = 0.5 * (jnp.sum(s * s, axis=1, keepdims=True) - ssq)
    linear = jnp.sum(lin_ref[...], axis=1, keepdims=True)
    h = jnp.maximum(h + b1_ref[...], 0.0)
    h = jnp.dot(h, w2_ref[...], preferred_element_type=jnp.float32)
    h = jnp.maximum(h + b2_ref[...], 0.0)
    z = jnp.dot(h, wfh_ref[...], preferred_element_type=jnp.float32)
    z = linear * scal_ref[0] + second * scal_ref[1] + z + scal_ref[2]
    o_ref[...] = 1.0 / (1.0 + jnp.exp(-z))


def _tc_head(e3, ling, W1f, b1, W2, b2, wfh, scal):
    grid = (B // BB,)
    return pl.pallas_call(
        _tc_body,
        grid=grid,
        in_specs=[
            pl.BlockSpec(memory_space=pltpu.SMEM),
            pl.BlockSpec((F, BB, D), lambda i: (0, i, 0)),
            pl.BlockSpec((BB, F), lambda i: (i, 0)),
            pl.BlockSpec((F, D, H1), lambda i: (0, 0, 0)),
            pl.BlockSpec((1, H1), lambda i: (0, 0)),
            pl.BlockSpec((H1, H2), lambda i: (0, 0)),
            pl.BlockSpec((1, H2), lambda i: (0, 0)),
            pl.BlockSpec((H2, 1), lambda i: (0, 0)),
        ],
        out_specs=pl.BlockSpec((BB, 1), lambda i: (i, 0)),
        out_shape=jax.ShapeDtypeStruct((B, 1), jnp.float32),
    )(scal, e3, ling, W1f, b1, W2, b2, wfh)


def kernel(x, emb_tables, lin_tables, W1, b1, W2, b2, Wf, bf):
    x = x.astype(jnp.int32)
    xw = x.reshape(NW, BPW, F).transpose(0, 2, 1)
    out_emb, out_lin = _sc_gather(xw, xw // LG, xw % LG, emb_tables,
                                  lin_tables.reshape(F, V // LG, LG))
    e3 = out_emb.reshape(F, B, D)
    ling = out_lin.reshape(NW, F, BPW).transpose(0, 2, 1).reshape(B, F)
    scal = jnp.stack([Wf[0, 0], Wf[1, 0], bf[0]])
    return _tc_head(e3, ling, W1.reshape(F, D, H1), b1.reshape(1, H1), W2,
                    b2.reshape(1, H2), Wf[2:, :], scal)


# speedup vs baseline: 1.8817x; 1.6055x over previous
import functools

import jax
import jax.numpy as jnp
from jax import lax
from jax.experimental import pallas as pl
from jax.experimental.pallas import tpu as pltpu
from jax.experimental.pallas import tpu_sc as plsc

F = 26
V = 100000
D = 32
B = 4096
H1 = 256
H2 = 128

NC = 2
NS = 16
NW = NC * NS
BPW = B // NW
LPW = BPW * F
CHUNK = BPW
LG = 32

_SC_MESH = plsc.VectorSubcoreMesh(core_axis_name="c", subcore_axis_name="s")


@functools.partial(
    pl.kernel,
    out_type=(
        jax.ShapeDtypeStruct((F, NW, D, CHUNK), jnp.float32),
        jax.ShapeDtypeStruct((NW, LPW), jnp.float32),
    ),
    mesh=_SC_MESH,
    scratch_types=(
        pltpu.VMEM((F, CHUNK), jnp.int32),
        pltpu.VMEM((F, CHUNK), jnp.int32),
        pltpu.VMEM((F, CHUNK), jnp.int32),
        pltpu.VMEM((2, D, CHUNK), jnp.float32),
        pltpu.VMEM((2, CHUNK, LG), jnp.float32),
        pltpu.VMEM((LPW,), jnp.float32),
        pltpu.SemaphoreType.DMA((2,)),
        pltpu.SemaphoreType.DMA((2,)),
    ),
    compiler_params=pltpu.CompilerParams(use_tc_tiling_on_sc=False,
                                         needs_layout_passes=False),
)
def _sc_gather(xw, xq, xr, embF, lin32, out_emb, out_lin, idx_v, idxq_v,
               rem_v, tchunk, lring, lin_v, sem_g, sem_lg):
    wid = lax.axis_index("s") * NC + lax.axis_index("c")
    pltpu.sync_copy(xw.at[wid], idx_v)
    pltpu.sync_copy(xq.at[wid], idxq_v)
    pltpu.sync_copy(xr.at[wid], rem_v)

    def do_field(f, slot):
        g_cps = [
            pltpu.async_copy(embF.at[f * D + d].at[idx_v.at[f]],
                             tchunk.at[slot, d], sem_g.at[slot])
            for d in range(D)
        ]
        l_cp = pltpu.async_copy(lin32.at[f].at[idxq_v.at[f]],
                                lring.at[slot], sem_lg.at[slot])
        return g_cps, l_cp

    def finish_field(f, slot, g_cps, l_cp):
        for cp in g_cps:
            cp.wait()
        pltpu.sync_copy(tchunk.at[slot], out_emb.at[f, wid])
        l_cp.wait()
        for g in range(CHUNK // 16):
            rows = lax.iota(jnp.int32, 16) + (g * 16)
            lanes = rem_v[f, pl.ds(g * 16, 16)]
            vals = plsc.load_gather(lring.at[slot], [rows, lanes])
            lin_v[pl.ds(f * CHUNK + g * 16, 16)] = vals

    def body(i, carry):
        f0 = i * 2
        f1 = f0 + 1
        g0, l0 = do_field(f0, 0)
        g1, l1 = do_field(f1, 1)
        finish_field(f0, 0, g0, l0)
        finish_field(f1, 1, g1, l1)
        return carry

    lax.fori_loop(0, F // 2, body, 0)
    pltpu.sync_copy(lin_v, out_lin.at[wid])


BB = BPW


def _tc_body(scal_ref, e_ref, lin_ref, w1_ref, b1_ref, w2_ref, b2_ref,
             wfh_ref, o_ref):
    cdim = (((0,), (0,)), ((), ()))
    s = jnp.zeros((D, BB), jnp.float32)
    esq = jnp.zeros((D, BB), jnp.float32)
    h = jnp.zeros((BB, H1), jnp.float32)
    for f in range(F):
        ef = e_ref[f, 0]
        s = s + ef
        esq = esq + ef * ef
        h = h + lax.dot_general(ef, w1_ref[f], cdim,
                                preferred_element_type=jnp.float32)
    ones = jnp.ones((D, 1), jnp.float32)
    sq_sum = lax.dot_general(s * s, ones, cdim,
                             preferred_element_type=jnp.float32)
    ssq = lax.dot_general(esq, ones, cdim,
                          preferred_element_type=jnp.float32)
    second = 0.5 * (sq_sum - ssq)
    linear = jnp.sum(lin_ref[...], axis=1, keepdims=True)
    h = jnp.maximum(h + b1_ref[...], 0.0)
    h = jnp.dot(h, w2_ref[...], preferred_element_type=jnp.float32)
    h = jnp.maximum(h + b2_ref[...], 0.0)
    z = jnp.dot(h, wfh_ref[...], preferred_element_type=jnp.float32)
    z = linear * scal_ref[0] + second * scal_ref[1] + z + scal_ref[2]
    o_ref[...] = 1.0 / (1.0 + jnp.exp(-z))


def _tc_head(e4, ling, W1f, b1, W2, b2, wfh, scal):
    grid = (NW,)
    return pl.pallas_call(
        _tc_body,
        grid=grid,
        in_specs=[
            pl.BlockSpec(memory_space=pltpu.SMEM),
            pl.BlockSpec((F, 1, D, CHUNK), lambda i: (0, i, 0, 0)),
            pl.BlockSpec((BB, F), lambda i: (i, 0)),
            pl.BlockSpec((F, D, H1), lambda i: (0, 0, 0)),
            pl.BlockSpec((1, H1), lambda i: (0, 0)),
            pl.BlockSpec((H1, H2), lambda i: (0, 0)),
            pl.BlockSpec((1, H2), lambda i: (0, 0)),
            pl.BlockSpec((H2, 1), lambda i: (0, 0)),
        ],
        out_specs=pl.BlockSpec((BB, 1), lambda i: (i, 0)),
        out_shape=jax.ShapeDtypeStruct((B, 1), jnp.float32),
    )(scal, e4, ling, W1f, b1, W2, b2, wfh)


def kernel(x, emb_tables, lin_tables, W1, b1, W2, b2, Wf, bf):
    x = x.astype(jnp.int32)
    xw = x.reshape(NW, BPW, F).transpose(0, 2, 1)
    embF = emb_tables.transpose(0, 2, 1).reshape(F * D, V)
    out_emb, out_lin = _sc_gather(xw, xw // LG, xw % LG, embF,
                                  lin_tables.reshape(F, V // LG, LG))
    ling = out_lin.reshape(NW, F, BPW).transpose(0, 2, 1).reshape(B, F)
    scal = jnp.stack([Wf[0, 0], Wf[1, 0], bf[0]])
    return _tc_head(out_emb, ling, W1.reshape(F, D, H1), b1.reshape(1, H1),
                    W2, b2.reshape(1, H2), Wf[2:, :], scal)


# speedup vs baseline: 1.8866x; 1.0026x over previous
import functools

import jax
import jax.numpy as jnp
from jax import lax
from jax.experimental import pallas as pl
from jax.experimental.pallas import tpu as pltpu
from jax.experimental.pallas import tpu_sc as plsc

F = 26
V = 100000
D = 32
B = 4096
H1 = 256
H2 = 128

NC = 2
NS = 16
NW = NC * NS
BPW = B // NW
LPW = BPW * F
CHUNK = BPW
LG = 32

_SC_MESH = plsc.VectorSubcoreMesh(core_axis_name="c", subcore_axis_name="s")


@functools.partial(
    pl.kernel,
    out_type=(
        jax.ShapeDtypeStruct((F, NW, D * CHUNK), jnp.float32),
        jax.ShapeDtypeStruct((NW, LPW), jnp.float32),
    ),
    mesh=_SC_MESH,
    scratch_types=(
        pltpu.VMEM((F, CHUNK), jnp.int32),
        pltpu.VMEM((F, CHUNK), jnp.int32),
        pltpu.VMEM((F, CHUNK), jnp.int32),
        pltpu.VMEM((2, D * CHUNK), jnp.int32),
        pltpu.VMEM((2, D * CHUNK), jnp.float32),
        pltpu.VMEM((2, CHUNK, LG), jnp.float32),
        pltpu.VMEM((LPW,), jnp.float32),
        pltpu.SemaphoreType.DMA((2,)),
        pltpu.SemaphoreType.DMA((2,)),
    ),
    compiler_params=pltpu.CompilerParams(use_tc_tiling_on_sc=False,
                                         needs_layout_passes=False),
)
def _sc_gather(xw, xq, xr, embL, lin32, out_emb, out_lin, idx_v, idxq_v,
               rem_v, idxe, tchunk, lring, lin_v, sem_g, sem_lg):
    wid = lax.axis_index("s") * NC + lax.axis_index("c")
    pltpu.sync_copy(xw.at[wid], idx_v)
    pltpu.sync_copy(xq.at[wid], idxq_v)
    pltpu.sync_copy(xr.at[wid], rem_v)

    def do_field(f, slot):
        regs = [idx_v[f, pl.ds(g * 16, 16)] + (f * (D * V))
                for g in range(CHUNK // 16)]
        for d in range(D):
            for g in range(CHUNK // 16):
                if d > 0:
                    regs[g] = regs[g] + V
                idxe[slot, pl.ds(d * CHUNK + g * 16, 16)] = regs[g]
        g_cp = pltpu.async_copy(embL.at[idxe.at[slot]], tchunk.at[slot],
                                sem_g.at[slot])
        l_cp = pltpu.async_copy(lin32.at[f].at[idxq_v.at[f]],
                                lring.at[slot], sem_lg.at[slot])
        return g_cp, l_cp

    def finish_field(f, slot, g_cp, l_cp):
        g_cp.wait()
        pltpu.sync_copy(tchunk.at[slot], out_emb.at[f, wid])
        l_cp.wait()
        for g in range(CHUNK // 16):
            rows = lax.iota(jnp.int32, 16) + (g * 16)
            lanes = rem_v[f, pl.ds(g * 16, 16)]
            vals = plsc.load_gather(lring.at[slot], [rows, lanes])
            lin_v[pl.ds(f * CHUNK + g * 16, 16)] = vals

    def body(i, carry):
        f0 = i * 2
        f1 = f0 + 1
        g0, l0 = do_field(f0, 0)
        g1, l1 = do_field(f1, 1)
        finish_field(f0, 0, g0, l0)
        finish_field(f1, 1, g1, l1)
        return carry

    lax.fori_loop(0, F // 2, body, 0)
    pltpu.sync_copy(lin_v, out_lin.at[wid])


BB = BPW


def _tc_body(scal_ref, e_ref, lin_ref, w1_ref, b1_ref, w2_ref, b2_ref,
             wfh_ref, o_ref):
    cdim = (((0,), (0,)), ((), ()))
    s = jnp.zeros((D, BB), jnp.float32)
    esq = jnp.zeros((D, BB), jnp.float32)
    h = jnp.zeros((BB, H1), jnp.float32)
    for f in range(F):
        ef = e_ref[f, 0]
        s = s + ef
        esq = esq + ef * ef
        h = h + lax.dot_general(ef, w1_ref[f], cdim,
                                preferred_element_type=jnp.float32)
    ones = jnp.ones((D, 1), jnp.float32)
    sq_sum = lax.dot_general(s * s, ones, cdim,
                             preferred_element_type=jnp.float32)
    ssq = lax.dot_general(esq, ones, cdim,
                          preferred_element_type=jnp.float32)
    second = 0.5 * (sq_sum - ssq)
    linear = jnp.sum(lin_ref[...], axis=1, keepdims=True)
    h = jnp.maximum(h + b1_ref[...], 0.0)
    h = jnp.dot(h, w2_ref[...], preferred_element_type=jnp.float32)
    h = jnp.maximum(h + b2_ref[...], 0.0)
    z = jnp.dot(h, wfh_ref[...], preferred_element_type=jnp.float32)
    z = linear * scal_ref[0] + second * scal_ref[1] + z + scal_ref[2]
    o_ref[...] = 1.0 / (1.0 + jnp.exp(-z))


def _tc_head(e4, ling, W1f, b1, W2, b2, wfh, scal):
    grid = (NW,)
    return pl.pallas_call(
        _tc_body,
        grid=grid,
        in_specs=[
            pl.BlockSpec(memory_space=pltpu.SMEM),
            pl.BlockSpec((F, 1, D, CHUNK), lambda i: (0, i, 0, 0)),
            pl.BlockSpec((BB, F), lambda i: (i, 0)),
            pl.BlockSpec((F, D, H1), lambda i: (0, 0, 0)),
            pl.BlockSpec((1, H1), lambda i: (0, 0)),
            pl.BlockSpec((H1, H2), lambda i: (0, 0)),
            pl.BlockSpec((1, H2), lambda i: (0, 0)),
            pl.BlockSpec((H2, 1), lambda i: (0, 0)),
        ],
        out_specs=pl.BlockSpec((BB, 1), lambda i: (i, 0)),
        out_shape=jax.ShapeDtypeStruct((B, 1), jnp.float32),
    )(scal, e4, ling, W1f, b1, W2, b2, wfh)


def kernel(x, emb_tables, lin_tables, W1, b1, W2, b2, Wf, bf):
    x = x.astype(jnp.int32)
    xw = x.reshape(NW, BPW, F).transpose(0, 2, 1)
    embL = emb_tables.transpose(0, 2, 1).reshape(F * D * V)
    out_emb, out_lin = _sc_gather(xw, xw // LG, xw % LG, embL,
                                  lin_tables.reshape(F, V // LG, LG))
    ling = out_lin.reshape(NW, F, BPW).transpose(0, 2, 1).reshape(B, F)
    scal = jnp.stack([Wf[0, 0], Wf[1, 0], bf[0]])
    e4 = out_emb.reshape(F, NW, D, CHUNK)
    return _tc_head(e4, ling, W1.reshape(F, D, H1), b1.reshape(1, H1),
                    W2, b2.reshape(1, H2), Wf[2:, :], scal)
